# baseline (speedup 1.0000x reference)
import jax
import jax.numpy as jnp
from jax import lax
from jax.experimental import pallas as pl
from jax.experimental.pallas import tpu as pltpu
from jax.experimental.pallas import tpu_sc as plsc

_R = 12800
_B = 4096
_BLK_R = 8
_BLK_B = 2048
_NW = 32
_JB = _B // _BLK_B
_NBLK = (_R // _BLK_R) * _JB // _NW
_DEPTH = 4


def _sc_add(x2t, epat):
    mesh = plsc.VectorSubcoreMesh(core_axis_name="core", subcore_axis_name="subcore")

    @pl.kernel(
        mesh=mesh,
        out_type=jax.ShapeDtypeStruct((_R, _B), jnp.float32),
        scratch_types=[
            pltpu.VMEM((_DEPTH, _BLK_R, _BLK_B), jnp.float32),
            pltpu.VMEM((_DEPTH, _BLK_R, 128), jnp.float32),
            pltpu.SemaphoreType.DMA((_DEPTH,)),
            pltpu.SemaphoreType.DMA((_DEPTH,)),
            pltpu.SemaphoreType.DMA((_DEPTH,)),
        ],
    )
    def k(x_hbm, e_hbm, o_hbm, xb, eb, s_in, s_e, s_out):
        cid = lax.axis_index("core")
        sid = lax.axis_index("subcore")
        w = sid * 2 + cid
        base = w * _NBLK

        def xsl(n):
            flat = base + n
            i = flat // _JB
            j = flat % _JB
            return (pl.ds(_BLK_R * i, _BLK_R), pl.ds(_BLK_B * j, _BLK_B))

        def esl(n):
            flat = base + n
            i = flat // _JB
            return (pl.ds(_BLK_R * i, _BLK_R), pl.ds(0, 128))

        def start_in(n, p):
            pltpu.async_copy(x_hbm.at[xsl(n)], xb.at[p], s_in.at[p])
            pltpu.async_copy(e_hbm.at[esl(n)], eb.at[p], s_e.at[p])

        for p in range(2):
            start_in(p, p)

        @pl.loop(0, _NBLK, step=_DEPTH)
        def _(nn):
            for p in range(_DEPTH):
                n = nn + p

                q = (p + 2) % _DEPTH

                @pl.when(n + 2 < _NBLK)
                def _():
                    @pl.when(n >= 2)
                    def _():
                        pltpu.make_async_copy(xb.at[q], o_hbm.at[xsl(n - 2)], s_out.at[q]).wait()
                    start_in(n + 2, q)

                pltpu.make_async_copy(x_hbm.at[xsl(n)], xb.at[p], s_in.at[p]).wait()
                pltpu.make_async_copy(e_hbm.at[esl(n)], eb.at[p], s_e.at[p]).wait()

                for r in range(_BLK_R):
                    evs = [eb.at[p, r, pl.ds(16 * kk, 16)][...] for kk in range(8)]

                    @plsc.parallel_loop(0, _BLK_B, step=128, unroll=2)
                    def _(g, evs=evs, r=r, p=p):
                        for kk in range(8):
                            plsc.addupdate(xb.at[p, r, pl.ds(g + 16 * kk, 16)], evs[kk])

                pltpu.async_copy(xb.at[p], o_hbm.at[xsl(n)], s_out.at[p])

        for p in range(_DEPTH):
            n_last = _NBLK - _DEPTH + p
            pltpu.make_async_copy(xb.at[p % _DEPTH], o_hbm.at[xsl(n_last)], s_out.at[n_last % _DEPTH]).wait()

    return k(x2t, epat)


def kernel(x, embedding):
    b, s, d = x.shape
    x2t = jnp.transpose(x, (1, 2, 0)).reshape(s * d, b)
    epat = jnp.broadcast_to(embedding.reshape(s * d, 1), (s * d, 128))
    out2 = _sc_add(x2t, epat)
    return jnp.transpose(out2.reshape(s, d, b), (2, 0, 1))
